# baseline (speedup 1.0000x reference)
import functools

import jax
import jax.numpy as jnp
from jax import lax
from jax.experimental import pallas as pl
from jax.experimental.pallas import tpu as pltpu
from jax.experimental.pallas import tpu_sc as plsc

B = 16384
N_FIELDS = 26
VOCAB = 100000
EMB = 32
NUM = 13
H1 = 256
H2 = 128
EMB_PAD = 896
NUM_PAD = 64

N_ROWS = B * N_FIELDS
NW = 32
B_PER_W = B // NW
NB = 16
CHUNK = NB * N_FIELDS
NCHUNK = B_PER_W // NB
NGRP = CHUNK // 16

_mesh = plsc.VectorSubcoreMesh(core_axis_name="c", subcore_axis_name="s")


@functools.partial(
    pl.kernel,
    mesh=_mesh,
    compiler_params=pltpu.CompilerParams(needs_layout_passes=False),
    out_type=jax.ShapeDtypeStruct((B, EMB_PAD), jnp.float32),
    scratch_types=[
        pltpu.VMEM((CHUNK,), jnp.int32),
        pltpu.VMEM((CHUNK,), jnp.int32),
        pltpu.VMEM((CHUNK, 128), jnp.float32),
        pltpu.VMEM((NB, EMB_PAD), jnp.float32),
        pltpu.SemaphoreType.DMA,
    ],
)
def _sc_gather(idx_hbm, table_hbm, out_hbm, idx_v, sidx_v, super_v, out_v, sem):
    wid = lax.axis_index("s") * 2 + lax.axis_index("c")
    lane = lax.iota(jnp.int32, 16)

    def body(c, carry):
        b0 = wid * B_PER_W + c * NB
        g0 = b0 * N_FIELDS
        pltpu.sync_copy(idx_hbm.at[pl.ds(g0, CHUNK)], idx_v)

        def shift_body(g, carry):
            sidx_v[pl.ds(g * 16, 16)] = lax.shift_right_logical(
                idx_v[pl.ds(g * 16, 16)], 2)
            return carry

        lax.fori_loop(0, NGRP, shift_body, 0)
        pltpu.async_copy(table_hbm.at[sidx_v], super_v, sem).wait()

        def compact_body(g, carry):
            rows = g * 16 + lane
            b_loc = lax.div(rows, N_FIELDS)
            fcol = (rows - b_loc * N_FIELDS) * EMB
            src0 = (idx_v[pl.ds(g * 16, 16)] & 3) * EMB
            for col in range(EMB):
                val = plsc.load_gather(super_v, [rows, src0 + col])
                plsc.store_scatter(out_v, [b_loc, fcol + col], val)
            return carry

        lax.fori_loop(0, NGRP, compact_body, 0)

        zero = jnp.zeros((16,), jnp.float32)

        def pad_body(b, carry):
            for k in range(4):
                out_v[b, pl.ds(832 + k * 16, 16)] = zero
            return carry

        lax.fori_loop(0, NB, pad_body, 0)
        pltpu.sync_copy(out_v, out_hbm.at[pl.ds(b0, NB)])
        return carry

    lax.fori_loop(0, NCHUNK, body, 0)


BM = 2048


def _mlp_body(emb_ref, num_ref, w1e_ref, w1n_ref, b1_ref, w2_ref, b2_ref, out_ref):
    h = jnp.dot(emb_ref[...], w1e_ref[...], preferred_element_type=jnp.float32)
    h = h + jnp.dot(num_ref[...], w1n_ref[...], preferred_element_type=jnp.float32)
    h = jnp.maximum(h + b1_ref[...], 0.0)
    o = jnp.dot(h, w2_ref[...], preferred_element_type=jnp.float32) + b2_ref[...]
    out_ref[...] = jnp.maximum(o, 0.0)


_mlp = pl.pallas_call(
    _mlp_body,
    grid=(B // BM,),
    in_specs=[
        pl.BlockSpec((BM, EMB_PAD), lambda i: (i, 0)),
        pl.BlockSpec((BM, NUM_PAD), lambda i: (i, 0)),
        pl.BlockSpec((EMB_PAD, H1), lambda i: (0, 0)),
        pl.BlockSpec((NUM_PAD, H1), lambda i: (0, 0)),
        pl.BlockSpec((1, H1), lambda i: (0, 0)),
        pl.BlockSpec((H1, H2), lambda i: (0, 0)),
        pl.BlockSpec((1, H2), lambda i: (0, 0)),
    ],
    out_specs=pl.BlockSpec((BM, H2), lambda i: (i, 0)),
    out_shape=jax.ShapeDtypeStruct((B, H2), jnp.float32),
)


def kernel(x_categorical, x_numerical, tables, bn_gamma, bn_beta, bn_mean, bn_var,
           W1, b1, W2, b2):
    x_cat = x_categorical.astype(jnp.int32)
    flat_idx = (x_cat + (jnp.arange(N_FIELDS, dtype=jnp.int32) * VOCAB)[None, :]
                ).reshape(-1)
    table128 = tables.reshape(N_FIELDS * VOCAB // 4, 128)
    emb = _sc_gather(flat_idx, table128)

    scale = bn_gamma * lax.rsqrt(bn_var + 1e-5)
    shift = bn_beta - bn_mean * scale
    W1e_T = jnp.zeros((EMB_PAD, H1), jnp.float32).at[:832].set(W1[:, :832].T)
    W1n = W1[:, 832:]
    W1n_T = (W1n * scale[None, :]).T
    W1n_T_pad = jnp.zeros((NUM_PAD, H1), jnp.float32).at[:NUM].set(W1n_T)
    b1_eff = (b1 + W1n @ shift).reshape(1, H1)
    x_num_pad = jnp.zeros((B, NUM_PAD), jnp.float32).at[:, :NUM].set(x_numerical)

    return _mlp(emb, x_num_pad, W1e_T, W1n_T_pad, b1_eff, W2.T, b2.reshape(1, H2))


# speedup vs baseline: 1.3713x; 1.3713x over previous
import functools

import jax
import jax.numpy as jnp
from jax import lax
from jax.experimental import pallas as pl
from jax.experimental.pallas import tpu as pltpu
from jax.experimental.pallas import tpu_sc as plsc

B = 16384
N_FIELDS = 26
VOCAB = 100000
EMB = 32
NUM = 13
H1 = 256
H2 = 128
EMB_DIM = 832
NUM_PAD = 64

NW = 32
B_PER_W = B // NW

_mesh = plsc.VectorSubcoreMesh(core_axis_name="c", subcore_axis_name="s")


@functools.partial(
    pl.kernel,
    mesh=_mesh,
    compiler_params=pltpu.CompilerParams(use_tc_tiling_on_sc=False),
    out_type=jax.ShapeDtypeStruct((B, EMB_DIM), jnp.float32),
    scratch_types=[
        pltpu.VMEM((B_PER_W,), jnp.int32),
        pltpu.VMEM((B_PER_W, EMB), jnp.float32),
        pltpu.SemaphoreType.DMA,
    ],
)
def _sc_gather(idxT_hbm, table_hbm, out_hbm, idx_v, rows_v, sem):
    wid = lax.axis_index("s") * 2 + lax.axis_index("c")
    b0 = wid * B_PER_W

    def body(f, carry):
        pltpu.sync_copy(idxT_hbm.at[f, pl.ds(b0, B_PER_W)], idx_v)
        pltpu.async_copy(table_hbm.at[f].at[idx_v], rows_v, sem).wait()
        pltpu.sync_copy(rows_v, out_hbm.at[pl.ds(b0, B_PER_W), pl.ds(f * EMB, EMB)])
        return carry

    lax.fori_loop(0, N_FIELDS, body, 0)


BM = 2048


def _mlp_body(emb_ref, num_ref, w1e_ref, w1n_ref, b1_ref, w2_ref, b2_ref, out_ref):
    h = jnp.dot(emb_ref[...], w1e_ref[...], preferred_element_type=jnp.float32)
    h = h + jnp.dot(num_ref[...], w1n_ref[...], preferred_element_type=jnp.float32)
    h = jnp.maximum(h + b1_ref[...], 0.0)
    o = jnp.dot(h, w2_ref[...], preferred_element_type=jnp.float32) + b2_ref[...]
    out_ref[...] = jnp.maximum(o, 0.0)


_mlp = pl.pallas_call(
    _mlp_body,
    grid=(B // BM,),
    in_specs=[
        pl.BlockSpec((BM, EMB_DIM), lambda i: (i, 0)),
        pl.BlockSpec((BM, NUM_PAD), lambda i: (i, 0)),
        pl.BlockSpec((EMB_DIM, H1), lambda i: (0, 0)),
        pl.BlockSpec((NUM_PAD, H1), lambda i: (0, 0)),
        pl.BlockSpec((1, H1), lambda i: (0, 0)),
        pl.BlockSpec((H1, H2), lambda i: (0, 0)),
        pl.BlockSpec((1, H2), lambda i: (0, 0)),
    ],
    out_specs=pl.BlockSpec((BM, H2), lambda i: (i, 0)),
    out_shape=jax.ShapeDtypeStruct((B, H2), jnp.float32),
)


def kernel(x_categorical, x_numerical, tables, bn_gamma, bn_beta, bn_mean, bn_var,
           W1, b1, W2, b2):
    x_catT = x_categorical.astype(jnp.int32).T
    emb = _sc_gather(x_catT, tables)

    scale = bn_gamma * lax.rsqrt(bn_var + 1e-5)
    shift = bn_beta - bn_mean * scale
    W1e_T = W1[:, :EMB_DIM].T
    W1n = W1[:, EMB_DIM:]
    W1n_T = (W1n * scale[None, :]).T
    W1n_T_pad = jnp.zeros((NUM_PAD, H1), jnp.float32).at[:NUM].set(W1n_T)
    b1_eff = (b1 + W1n @ shift).reshape(1, H1)
    x_num_pad = jnp.zeros((B, NUM_PAD), jnp.float32).at[:, :NUM].set(x_numerical)

    return _mlp(emb, x_num_pad, W1e_T, W1n_T_pad, b1_eff, W2.T, b2.reshape(1, H2))
